# Initial kernel scaffold; baseline (speedup 1.0000x reference)
import jax
import jax.numpy as jnp
from jax.experimental import pallas as pl


def kernel(x, ln_g, ln_b, w1_v, w1_g, b1, w2_v, w2_g, b2, w3_v, w3_g, b3, expert_bias):
    raise NotImplementedError("write your pallas kernel here")


# speedup vs baseline: 1.1758x; 1.1758x over previous
import jax
import jax.numpy as jnp
from jax.experimental import pallas as pl
from jax.experimental.pallas import tpu as pltpu

TOKENS = 8192
D_MODEL = 2048
HIDDEN = 4096
EXPERTS = 16

TB = 1024
HB = 256
T = TOKENS // TB
H = HIDDEN // HB

_BF = jnp.bfloat16
_DN = (((1,), (1,)), ((), ()))


def _router_kernel(x_ref, mu_ref, var_ref, ln_g_ref, ln_b_ref,
                   w1v_ref, w1g_ref, n1_ref, b1_ref,
                   w2v_ref, w2g_ref, n2_ref, b2_ref,
                   w3v_ref, w3g_ref, n3_ref, b3_ref, eb_ref,
                   idx_ref, prob_ref, usage_ref,
                   xn_ref, sw_ref, w3n_ref):
    t = pl.program_id(0)
    h = pl.program_id(1)

    @pl.when(h == 0)
    def _prologue():
        xb = x_ref[...]
        mu = mu_ref[...][:, None]
        var = var_ref[...][:, None]
        xn = (xb - mu) / jnp.sqrt(var + 1e-5)
        xn = xn * ln_g_ref[...][None, :] + ln_b_ref[...][None, :]
        xn_ref[...] = xn.astype(_BF)
        v3 = w3v_ref[...]
        w3n = v3 * w3g_ref[...][:, None] / (n3_ref[...][:, None] + 1e-12)
        w3n_ref[...] = w3n.astype(_BF)

    xn = xn_ref[...]
    v1 = w1v_ref[...]
    v2 = w2v_ref[...]
    w1n = v1 * w1g_ref[...][:, None] / (n1_ref[...][:, None] + 1e-12)
    w2n = v2 * w2g_ref[...][:, None] / (n2_ref[...][:, None] + 1e-12)

    h1 = jax.lax.dot_general(xn, w1n.astype(_BF), _DN,
                             preferred_element_type=jnp.float32)
    h1 = h1 + b1_ref[...][None, :]
    h2 = jax.lax.dot_general(xn, w2n.astype(_BF), _DN,
                             preferred_element_type=jnp.float32)
    h2 = h2 + b2_ref[...][None, :]
    sw_ref[:, pl.ds(h * HB, HB)] = (h1 * jax.nn.sigmoid(h2)).astype(_BF)

    @pl.when(h == H - 1)
    def _route():
        logits = jax.lax.dot_general(sw_ref[...], w3n_ref[...], _DN,
                                     preferred_element_type=jnp.float32)
        logits = logits + b3_ref[...][None, :] + eb_ref[...][None, :]
        cols = jax.lax.broadcasted_iota(jnp.int32, logits.shape, 1)
        m1 = jnp.max(logits, axis=1, keepdims=True)
        i1 = jnp.min(jnp.where(logits == m1, cols, EXPERTS),
                     axis=1, keepdims=True)
        masked = jnp.where(cols == i1, -jnp.inf, logits)
        m2 = jnp.max(masked, axis=1, keepdims=True)
        i2 = jnp.min(jnp.where(masked == m2, cols, EXPERTS),
                     axis=1, keepdims=True)
        d = m1 - m2
        idx_ref[...] = jnp.concatenate([i1, i2], axis=1)
        prob_ref[...] = jnp.concatenate(
            [jax.nn.sigmoid(d), jax.nn.sigmoid(-d)], axis=1)
        one = ((cols == i1) | (cols == i2)).astype(jnp.float32)
        part = jnp.sum(one, axis=0, keepdims=True)

        @pl.when(t == 0)
        def _init_usage():
            usage_ref[...] = part

        @pl.when(t != 0)
        def _acc_usage():
            usage_ref[...] += part


def kernel(x, ln_g, ln_b, w1_v, w1_g, b1, w2_v, w2_g, b2,
           w3_v, w3_g, b3, expert_bias):
    mu = jnp.mean(x, axis=-1, keepdims=True)
    var = jnp.mean((x - mu) ** 2, axis=-1, keepdims=True)
    n1 = jnp.sqrt(jnp.sum(w1_v * w1_v, axis=1, keepdims=True) + 0.0)
    n2 = jnp.sqrt(jnp.sum(w2_v * w2_v, axis=1, keepdims=True) + 0.0)
    n3 = jnp.sqrt(jnp.sum(w3_v * w3_v, axis=1, keepdims=True) + 0.0)

    out_shapes = (
        jax.ShapeDtypeStruct((TOKENS, 2), jnp.int32),
        jax.ShapeDtypeStruct((TOKENS, 2), jnp.float32),
        jax.ShapeDtypeStruct((1, EXPERTS), jnp.float32),
    )
    grid = (T, H)
    in_specs = [
        pl.BlockSpec((TB, D_MODEL), lambda t, h: (t, 0)),
        pl.BlockSpec((TB,), lambda t, h: (t,)),
        pl.BlockSpec((TB,), lambda t, h: (t,)),
        pl.BlockSpec((D_MODEL,), lambda t, h: (0,)),
        pl.BlockSpec((D_MODEL,), lambda t, h: (0,)),
        pl.BlockSpec((HB, D_MODEL), lambda t, h: (h, 0)),
        pl.BlockSpec((HB,), lambda t, h: (h,)),
        pl.BlockSpec((HB,), lambda t, h: (h,)),
        pl.BlockSpec((HB,), lambda t, h: (h,)),
        pl.BlockSpec((HB, D_MODEL), lambda t, h: (h, 0)),
        pl.BlockSpec((HB,), lambda t, h: (h,)),
        pl.BlockSpec((HB,), lambda t, h: (h,)),
        pl.BlockSpec((HB,), lambda t, h: (h,)),
        pl.BlockSpec((EXPERTS, HIDDEN), lambda t, h: (0, 0)),
        pl.BlockSpec((EXPERTS,), lambda t, h: (0,)),
        pl.BlockSpec((EXPERTS,), lambda t, h: (0,)),
        pl.BlockSpec((EXPERTS,), lambda t, h: (0,)),
        pl.BlockSpec((EXPERTS,), lambda t, h: (0,)),
    ]
    out_specs = (
        pl.BlockSpec((TB, 2), lambda t, h: (t, 0)),
        pl.BlockSpec((TB, 2), lambda t, h: (t, 0)),
        pl.BlockSpec((1, EXPERTS), lambda t, h: (0, 0)),
    )
    scratch_shapes = [
        pltpu.VMEM((TB, D_MODEL), _BF),
        pltpu.VMEM((TB, HIDDEN), _BF),
        pltpu.VMEM((EXPERTS, HIDDEN), _BF),
    ]
    idx, probs, usage = pl.pallas_call(
        _router_kernel,
        grid=grid,
        in_specs=in_specs,
        out_specs=out_specs,
        out_shape=out_shapes,
        scratch_shapes=scratch_shapes,
        compiler_params=pltpu.CompilerParams(
            dimension_semantics=("arbitrary", "arbitrary"),
        ),
    )(x, mu.reshape(TOKENS), var.reshape(TOKENS), ln_g, ln_b,
      w1_v, w1_g, n1.reshape(HIDDEN), b1,
      w2_v, w2_g, n2.reshape(HIDDEN), b2,
      w3_v, w3_g, n3.reshape(EXPERTS), b3, expert_bias)
    return (idx, probs, usage.reshape(EXPERTS))
